# Initial kernel scaffold; baseline (speedup 1.0000x reference)
import jax
import jax.numpy as jnp
from jax.experimental import pallas as pl


def kernel(fea, edge_index, w_in1, w_in2, incep_ws, w_out1, w_out2):
    raise NotImplementedError("write your pallas kernel here")


# speedup vs baseline: 15.4319x; 15.4319x over previous
import functools

import jax
import jax.numpy as jnp
from jax import lax
from jax.experimental import pallas as pl
from jax.experimental.pallas import tpu as pltpu
from jax.experimental.pallas import tpu_sc as plsc

N_NODES = 10000
NP = 10240
E_EDGES = 320000
EP = 327680
EDGES_PER_W = 10240
IDX_ROWS_PER_W = 80
NCHUNK = 10
GROUPS = 8
CHUNK_E = GROUPS * 128
DUMP_ROW = 10016
ROWS_PER_S = 640
RB = 1024

_mesh = plsc.VectorSubcoreMesh(core_axis_name="c", subcore_axis_name="s")


def _make_agg(F):
    @functools.partial(
        pl.kernel,
        out_type=jax.ShapeDtypeStruct((2, NP, F), jnp.float32),
        mesh=_mesh,
        scratch_types=[
            pltpu.VMEM((GROUPS, 128), jnp.int32),
            pltpu.VMEM((GROUPS, 128), jnp.int32),
            pltpu.VMEM((CHUNK_E, F), jnp.float32),
            pltpu.VMEM((ROWS_PER_S, F), jnp.float32),
            pltpu.VMEM_SHARED((NP, F), jnp.float32),
            pltpu.SemaphoreType.DMA,
        ],
        compiler_params=pltpu.CompilerParams(use_tc_tiling_on_sc=False),
    )
    def agg(h, srcr, dstr, zrows, out, srci, dsti, rows, stage, acc, sem):
        c = lax.axis_index("c")
        s = lax.axis_index("s")
        wid = c * 16 + s
        pltpu.sync_copy(zrows, stage)
        pltpu.sync_copy(stage, acc.at[pl.ds(s * ROWS_PER_S, ROWS_PER_S)])
        plsc.subcore_barrier()

        def chunk_body(ci, carry):
            r0 = wid * IDX_ROWS_PER_W + ci * GROUPS
            pltpu.sync_copy(srcr.at[pl.ds(r0, GROUPS)], srci)
            pltpu.sync_copy(dstr.at[pl.ds(r0, GROUPS)], dsti)
            for j in range(GROUPS):
                pltpu.async_copy(
                    h.at[srci.at[j]], rows.at[pl.ds(j * 128, 128)], sem
                ).wait()
            for j in range(GROUPS):
                pltpu.sync_copy(
                    rows.at[pl.ds(j * 128, 128)], acc.at[dsti.at[j]], add=True
                )
            return carry

        lax.fori_loop(0, NCHUNK, chunk_body, 0)
        plsc.subcore_barrier()
        pltpu.sync_copy(acc.at[pl.ds(s * ROWS_PER_S, ROWS_PER_S)], stage)
        pltpu.sync_copy(stage, out.at[c, pl.ds(s * ROWS_PER_S, ROWS_PER_S)])

    return agg


_agg16 = _make_agg(16)
_agg32 = _make_agg(32)


def _row_specs(shape, ncols):
    del shape
    return pl.BlockSpec((RB, ncols), lambda i: (i, 0))


def _pair_spec(ncols):
    return pl.BlockSpec((2, RB, ncols), lambda i: (0, i, 0))


def _full_spec(shape):
    nd = len(shape)
    return pl.BlockSpec(shape, lambda i: (0,) * nd)


def _tc_call(body, in_arrays, in_specs, out_shapes, out_specs):
    return pl.pallas_call(
        body,
        grid=(NP // RB,),
        in_specs=in_specs,
        out_specs=out_specs,
        out_shape=out_shapes,
    )(*in_arrays)


def _prep(fea_p, w_in1, ws_flat, w_out1):
    nws = len(ws_flat)

    def body(fea_ref, w1_ref, *refs):
        ws_refs = refs[:nws]
        wo1_ref = refs[nws]
        m1_ref = refs[nws + 1]
        d_ref = refs[nws + 2]
        m1_ref[...] = jnp.dot(
            fea_ref[...], w1_ref[...], preferred_element_type=jnp.float32
        )
        d_ref[0] = wo1_ref[0:16, :]
        wi = 0
        for k in range(6):
            C = ws_refs[wi][...]
            wi += 1
            for _ in range(k):
                C = jnp.dot(C, ws_refs[wi][...], preferred_element_type=jnp.float32)
                wi += 1
            d_ref[k + 1] = jnp.dot(
                C,
                wo1_ref[16 + 32 * k : 48 + 32 * k, :],
                preferred_element_type=jnp.float32,
            )

    in_specs = (
        [_row_specs((NP, 128), 128), _full_spec((128, 32))]
        + [_full_spec(w.shape) for w in ws_flat]
        + [_full_spec((208, 32))]
    )
    out_shapes = (
        jax.ShapeDtypeStruct((NP, 32), jnp.float32),
        jax.ShapeDtypeStruct((7, 16, 32), jnp.float32),
    )
    out_specs = (_row_specs((NP, 32), 32), _full_spec((7, 16, 32)))
    return _tc_call(body, [fea_p, w_in1] + ws_flat + [w_out1], in_specs, out_shapes, out_specs)


def _relu_mm(a_pair, w, fin, fout):
    def body(a_ref, w_ref, o_ref):
        h = jax.nn.relu(a_ref[0] + a_ref[1])
        o_ref[...] = jnp.dot(h, w_ref[...], preferred_element_type=jnp.float32)

    return _tc_call(
        body,
        [a_pair, w],
        [_pair_spec(fin), _full_spec((fin, fout))],
        jax.ShapeDtypeStruct((NP, fout), jnp.float32),
        _row_specs((NP, fout), fout),
    )


def _x_acc0(b_pair, d_mats):
    def body(b_ref, d_ref, x_ref, acc_ref):
        x = b_ref[0] + b_ref[1]
        x_ref[...] = x
        acc_ref[...] = jnp.dot(x, d_ref[0], preferred_element_type=jnp.float32)

    return _tc_call(
        body,
        [b_pair, d_mats],
        [_pair_spec(16), _full_spec((7, 16, 32))],
        (
            jax.ShapeDtypeStruct((NP, 16), jnp.float32),
            jax.ShapeDtypeStruct((NP, 32), jnp.float32),
        ),
        (_row_specs((NP, 16), 16), _row_specs((NP, 32), 32)),
    )


def _chain_step(q_pair, acc_in, d_mats, k):
    def body(q_ref, acc_ref, d_ref, p_ref, out_ref):
        p = q_ref[0] + q_ref[1]
        p_ref[...] = p
        out_ref[...] = acc_ref[...] + jnp.dot(
            p, d_ref[k], preferred_element_type=jnp.float32
        )

    return _tc_call(
        body,
        [q_pair, acc_in, d_mats],
        [_pair_spec(16), _row_specs((NP, 32), 32), _full_spec((7, 16, 32))],
        (
            jax.ShapeDtypeStruct((NP, 16), jnp.float32),
            jax.ShapeDtypeStruct((NP, 32), jnp.float32),
        ),
        (_row_specs((NP, 16), 16), _row_specs((NP, 32), 32)),
    )


def _finish(u_pair):
    def body(u_ref, o_ref):
        o = u_ref[0] + u_ref[1]
        nrm = jnp.sqrt(jnp.sum(o * o, axis=1, keepdims=True))
        o_ref[...] = o / jnp.maximum(nrm, 1e-12)

    return _tc_call(
        body,
        [u_pair],
        [_pair_spec(16)],
        jax.ShapeDtypeStruct((NP, 16), jnp.float32),
        _row_specs((NP, 16), 16),
    )


def kernel(fea, edge_index, w_in1, w_in2, incep_ws, w_out1, w_out2):
    src = jnp.concatenate(
        [edge_index[0], jnp.zeros((EP - E_EDGES,), jnp.int32)]
    ).reshape(EP // 128, 128)
    dst = jnp.concatenate(
        [edge_index[1], jnp.full((EP - E_EDGES,), DUMP_ROW, jnp.int32)]
    ).reshape(EP // 128, 128)
    fea_p = jnp.concatenate(
        [fea, jnp.zeros((NP - N_NODES, fea.shape[1]), jnp.float32)]
    )
    z16 = jnp.zeros((ROWS_PER_S, 16), jnp.float32)
    z32 = jnp.zeros((ROWS_PER_S, 32), jnp.float32)
    ws_flat = [w for ws in incep_ws for w in ws]

    m1, d_mats = _prep(fea_p, w_in1, ws_flat, w_out1)
    a = _agg32(m1, src, dst, z32)
    m2 = _relu_mm(a, w_in2, 32, 16)
    b = _agg16(m2, src, dst, z16)
    x, acc = _x_acc0(b, d_mats)

    q = _agg16(x, src, dst, z16)
    for k in range(1, 7):
        p, acc = _chain_step(q, acc, d_mats, k)
        if k < 6:
            q = _agg16(p, src, dst, z16)

    r = _agg32(acc, src, dst, z32)
    m3 = _relu_mm(r, w_out2, 32, 16)
    u = _agg16(m3, src, dst, z16)
    out = _finish(u)
    return out[:N_NODES]


# speedup vs baseline: 20.8146x; 1.3488x over previous
import functools

import jax
import jax.numpy as jnp
from jax import lax
from jax.experimental import pallas as pl
from jax.experimental.pallas import tpu as pltpu
from jax.experimental.pallas import tpu_sc as plsc

N_NODES = 10000
NP = 10240
E_EDGES = 320000
EP = 327680
EDGES_PER_W = 10240
IDX_ROWS_PER_W = 80
NCHUNK = 10
GROUPS = 8
CHUNK_E = GROUPS * 128
DUMP_ROW = 10016
ROWS_PER_S = 640
RB = 1024

_mesh = plsc.VectorSubcoreMesh(core_axis_name="c", subcore_axis_name="s")


def _make_agg(F):
    cr = 20 if F == 16 else 10
    nch = IDX_ROWS_PER_W // cr
    ce = cr * 128

    @functools.partial(
        pl.kernel,
        out_type=jax.ShapeDtypeStruct((2, NP, F), jnp.float32),
        mesh=_mesh,
        scratch_types=[
            pltpu.VMEM((nch, ce), jnp.int32),
            pltpu.VMEM((nch, ce), jnp.int32),
            pltpu.VMEM((2, ce, F), jnp.float32),
            pltpu.VMEM_SHARED((NP, F), jnp.float32),
            pltpu.SemaphoreType.DMA,
            pltpu.SemaphoreType.DMA,
        ],
        compiler_params=pltpu.CompilerParams(use_tc_tiling_on_sc=False),
    )
    def agg(h, srcr, dstr, zrows, out, srci, dsti, rows, acc, sem0, sem1):
        c = lax.axis_index("c")
        s = lax.axis_index("s")
        wid = c * 16 + s
        sems = (sem0, sem1)
        pltpu.sync_copy(zrows, rows.at[0, pl.ds(0, ROWS_PER_S)])
        pltpu.sync_copy(
            rows.at[0, pl.ds(0, ROWS_PER_S)],
            acc.at[pl.ds(s * ROWS_PER_S, ROWS_PER_S)],
        )
        base = wid * EDGES_PER_W
        for k in range(nch):
            pltpu.sync_copy(srcr.at[pl.ds(base + k * ce, ce)], srci.at[k])
            pltpu.sync_copy(dstr.at[pl.ds(base + k * ce, ce)], dsti.at[k])
        plsc.subcore_barrier()

        copies = [
            pltpu.make_async_copy(h.at[srci.at[k]], rows.at[k % 2], sems[k % 2])
            for k in range(nch)
        ]
        copies[0].start()
        for k in range(nch):
            if k + 1 < nch:
                copies[k + 1].start()
            copies[k].wait()
            pltpu.sync_copy(rows.at[k % 2], acc.at[dsti.at[k]], add=True)
        plsc.subcore_barrier()
        pltpu.sync_copy(
            acc.at[pl.ds(s * ROWS_PER_S, ROWS_PER_S)],
            rows.at[0, pl.ds(0, ROWS_PER_S)],
        )
        pltpu.sync_copy(
            rows.at[0, pl.ds(0, ROWS_PER_S)],
            out.at[c, pl.ds(s * ROWS_PER_S, ROWS_PER_S)],
        )

    return agg


_agg16 = _make_agg(16)
_agg32 = _make_agg(32)


def _row_specs(shape, ncols):
    del shape
    return pl.BlockSpec((RB, ncols), lambda i: (i, 0))


def _pair_spec(ncols):
    return pl.BlockSpec((2, RB, ncols), lambda i: (0, i, 0))


def _full_spec(shape):
    nd = len(shape)
    return pl.BlockSpec(shape, lambda i: (0,) * nd)


def _tc_call(body, in_arrays, in_specs, out_shapes, out_specs):
    return pl.pallas_call(
        body,
        grid=(NP // RB,),
        in_specs=in_specs,
        out_specs=out_specs,
        out_shape=out_shapes,
    )(*in_arrays)


def _prep(fea_p, w_in1, ws_flat, w_out1):
    nws = len(ws_flat)

    def body(fea_ref, w1_ref, *refs):
        ws_refs = refs[:nws]
        wo1_ref = refs[nws]
        m1_ref = refs[nws + 1]
        d_ref = refs[nws + 2]
        m1_ref[...] = jnp.dot(
            fea_ref[...], w1_ref[...], preferred_element_type=jnp.float32
        )
        d_ref[0] = wo1_ref[0:16, :]
        wi = 0
        for k in range(6):
            C = ws_refs[wi][...]
            wi += 1
            for _ in range(k):
                C = jnp.dot(C, ws_refs[wi][...], preferred_element_type=jnp.float32)
                wi += 1
            d_ref[k + 1] = jnp.dot(
                C,
                wo1_ref[16 + 32 * k : 48 + 32 * k, :],
                preferred_element_type=jnp.float32,
            )

    in_specs = (
        [_row_specs((NP, 128), 128), _full_spec((128, 32))]
        + [_full_spec(w.shape) for w in ws_flat]
        + [_full_spec((208, 32))]
    )
    out_shapes = (
        jax.ShapeDtypeStruct((NP, 32), jnp.float32),
        jax.ShapeDtypeStruct((7, 16, 32), jnp.float32),
    )
    out_specs = (_row_specs((NP, 32), 32), _full_spec((7, 16, 32)))
    return _tc_call(body, [fea_p, w_in1] + ws_flat + [w_out1], in_specs, out_shapes, out_specs)


def _relu_mm(a_pair, w, fin, fout):
    def body(a_ref, w_ref, o_ref):
        h = jax.nn.relu(a_ref[0] + a_ref[1])
        o_ref[...] = jnp.dot(h, w_ref[...], preferred_element_type=jnp.float32)

    return _tc_call(
        body,
        [a_pair, w],
        [_pair_spec(fin), _full_spec((fin, fout))],
        jax.ShapeDtypeStruct((NP, fout), jnp.float32),
        _row_specs((NP, fout), fout),
    )


def _x_acc0(b_pair, d_mats):
    def body(b_ref, d_ref, x_ref, acc_ref):
        x = b_ref[0] + b_ref[1]
        x_ref[...] = x
        acc_ref[...] = jnp.dot(x, d_ref[0], preferred_element_type=jnp.float32)

    return _tc_call(
        body,
        [b_pair, d_mats],
        [_pair_spec(16), _full_spec((7, 16, 32))],
        (
            jax.ShapeDtypeStruct((NP, 16), jnp.float32),
            jax.ShapeDtypeStruct((NP, 32), jnp.float32),
        ),
        (_row_specs((NP, 16), 16), _row_specs((NP, 32), 32)),
    )


def _chain_step(q_pair, acc_in, d_mats, k):
    def body(q_ref, acc_ref, d_ref, p_ref, out_ref):
        p = q_ref[0] + q_ref[1]
        p_ref[...] = p
        out_ref[...] = acc_ref[...] + jnp.dot(
            p, d_ref[k], preferred_element_type=jnp.float32
        )

    return _tc_call(
        body,
        [q_pair, acc_in, d_mats],
        [_pair_spec(16), _row_specs((NP, 32), 32), _full_spec((7, 16, 32))],
        (
            jax.ShapeDtypeStruct((NP, 16), jnp.float32),
            jax.ShapeDtypeStruct((NP, 32), jnp.float32),
        ),
        (_row_specs((NP, 16), 16), _row_specs((NP, 32), 32)),
    )


def _finish(u_pair):
    def body(u_ref, o_ref):
        o = u_ref[0] + u_ref[1]
        nrm = jnp.sqrt(jnp.sum(o * o, axis=1, keepdims=True))
        o_ref[...] = o / jnp.maximum(nrm, 1e-12)

    return _tc_call(
        body,
        [u_pair],
        [_pair_spec(16)],
        jax.ShapeDtypeStruct((NP, 16), jnp.float32),
        _row_specs((NP, 16), 16),
    )


def kernel(fea, edge_index, w_in1, w_in2, incep_ws, w_out1, w_out2):
    src = jnp.concatenate([edge_index[0], jnp.zeros((EP - E_EDGES,), jnp.int32)])
    dst = jnp.concatenate(
        [edge_index[1], jnp.full((EP - E_EDGES,), DUMP_ROW, jnp.int32)]
    )
    fea_p = jnp.concatenate(
        [fea, jnp.zeros((NP - N_NODES, fea.shape[1]), jnp.float32)]
    )
    z16 = jnp.zeros((ROWS_PER_S, 16), jnp.float32)
    z32 = jnp.zeros((ROWS_PER_S, 32), jnp.float32)
    ws_flat = [w for ws in incep_ws for w in ws]

    m1, d_mats = _prep(fea_p, w_in1, ws_flat, w_out1)
    a = _agg32(m1, src, dst, z32)
    m2 = _relu_mm(a, w_in2, 32, 16)
    b = _agg16(m2, src, dst, z16)
    x, acc = _x_acc0(b, d_mats)

    q = _agg16(x, src, dst, z16)
    for k in range(1, 7):
        p, acc = _chain_step(q, acc, d_mats, k)
        if k < 6:
            q = _agg16(p, src, dst, z16)

    r = _agg32(acc, src, dst, z32)
    m3 = _relu_mm(r, w_out2, 32, 16)
    u = _agg16(m3, src, dst, z16)
    out = _finish(u)
    return out[:N_NODES]
